# baseline (speedup 1.0000x reference)
import functools

import numpy as np
import jax
import jax.numpy as jnp
from jax import lax
from jax.experimental import pallas as pl
from jax.experimental.pallas import tpu as pltpu
from jax.experimental.pallas import tpu_sc as plsc

_FIELD_DIMS = [100000] * 26
_OFFSET_NP = np.array([0] + list(np.cumsum(_FIELD_DIMS))[:-1], dtype=np.int32)

_B = 4096
_F = 26
_D = 64
_LANES = 16
_NW = 32
_RPW = _B // _NW
_GROUP_ROWS = 4
_GIDX = _GROUP_ROWS * _F
_NG = _RPW // _GROUP_ROWS
_CHUNK = _RPW * _F


def _body(x_hbm, offs_hbm, w_hbm, bias_hbm, out_hbm,
          xv, offsv, idxv, biasv, buf0, buf1, outv, sem0, sem1):
    cid = lax.axis_index("c")
    sid = lax.axis_index("s")
    wid = sid * 2 + cid

    base = pl.multiple_of(wid * _CHUNK, _CHUNK)
    pltpu.sync_copy(x_hbm.at[pl.ds(base, _CHUNK)], xv)
    pltpu.sync_copy(offs_hbm, offsv)
    pltpu.sync_copy(bias_hbm, biasv)

    for i in range(_CHUNK // _LANES):
        sl = pl.ds(i * _LANES, _LANES)
        idxv[sl] = xv[sl] + offsv[sl]

    def issue(g, buf, sem):
        off = pl.multiple_of(g * _GIDX, 8)
        pltpu.async_copy(w_hbm.at[idxv.at[pl.ds(off, _GIDX)]], buf, sem)

    def wait(buf, sem):
        pltpu.make_async_copy(w_hbm.at[pl.ds(0, _GIDX)], buf, sem).wait()

    def accumulate(g, buf):
        for r in range(_GROUP_ROWS):
            row = g * _GROUP_ROWS + r
            accs = [biasv[pl.ds(k * _LANES, _LANES)] for k in range(_D // _LANES)]
            for f in range(_F):
                j = r * _F + f
                for k in range(_D // _LANES):
                    accs[k] = accs[k] + buf[j, pl.ds(k * _LANES, _LANES)]
            for k in range(_D // _LANES):
                outv[row, pl.ds(k * _LANES, _LANES)] = accs[k]

    issue(0, buf0, sem0)
    issue(1, buf1, sem1)

    def outer(g2, carry):
        g = g2 * 2
        wait(buf0, sem0)
        accumulate(g, buf0)
        issue(g + 2, buf0, sem0)
        wait(buf1, sem1)
        accumulate(g + 1, buf1)
        issue(g + 3, buf1, sem1)
        return carry

    lax.fori_loop(0, _NG // 2 - 1, outer, 0)

    wait(buf0, sem0)
    accumulate(_NG - 2, buf0)
    wait(buf1, sem1)
    accumulate(_NG - 1, buf1)

    obase = pl.multiple_of(wid * _RPW, _RPW)
    pltpu.sync_copy(outv, out_hbm.at[pl.ds(obase, _RPW)])


@jax.jit
def _fieldlinear_sc(xf, offs, weight, bias):
    mesh = plsc.VectorSubcoreMesh(core_axis_name="c", subcore_axis_name="s")
    kern = functools.partial(
        pl.kernel,
        out_type=jax.ShapeDtypeStruct((_B, _D), jnp.float32),
        mesh=mesh,
        compiler_params=pltpu.CompilerParams(use_tc_tiling_on_sc=False),
        scratch_types=[
            pltpu.VMEM((_CHUNK,), jnp.int32),
            pltpu.VMEM((_CHUNK,), jnp.int32),
            pltpu.VMEM((_CHUNK,), jnp.int32),
            pltpu.VMEM((_D,), jnp.float32),
            pltpu.VMEM((_GIDX, _D), jnp.float32),
            pltpu.VMEM((_GIDX, _D), jnp.float32),
            pltpu.VMEM((_RPW, _D), jnp.float32),
            pltpu.SemaphoreType.DMA,
            pltpu.SemaphoreType.DMA,
        ],
    )(_body)
    return kern(xf, offs, weight, bias)


def kernel(x, weight, bias):
    xf = x.reshape(-1)
    offs = jnp.asarray(np.tile(_OFFSET_NP, _RPW))
    return _fieldlinear_sc(xf, offs, weight, bias)


# speedup vs baseline: 1.1719x; 1.1719x over previous
import functools

import jax
import jax.numpy as jnp
from jax import lax
from jax.experimental import pallas as pl
from jax.experimental.pallas import tpu as pltpu
from jax.experimental.pallas import tpu_sc as plsc

_B = 4096
_F = 26
_D = 64
_V = 100000
_N = _F * _V
_L = 16
_CHK = 12544
_NK = 8
_TAIL0 = 2599936
_LAST7 = _TAIL0 - _CHK


def _body(x3_hbm, wt_hbm, bias_hbm, tail4_hbm, out5_hbm,
          xfv, buf, acc, tmp, macc, tailv, biasv, mbuf):
    cid = lax.axis_index("c")
    sid = lax.axis_index("s")
    k = sid % 8
    bh = sid // 8

    pltpu.sync_copy(bias_hbm, biasv)

    zero16 = jnp.zeros((_L,), jnp.float32)

    def pq(j):
        p = j // 8
        q0 = pl.multiple_of((j % 8) * _L, _L)
        return p, q0

    def gather_block(j, shift, lo, hi, src, carry):
        p, q0 = pq(j)
        xa = xfv[p, pl.ds(q0, _L)]
        idxl = xa + shift
        valid = (idxl >= lo) & (idxl < hi)
        idc = jnp.clip(idxl, 0, hi - 1)
        for r in range(8):
            rvec = jnp.full((_L,), r, dtype=jnp.int32)
            v = plsc.load_gather(src, [rvec, idc])
            vm = jnp.where(valid, v, 0.0)
            plsc.addupdate(acc.at[r, p, pl.ds(q0, _L)], vm)
        return carry

    for tr in range(4):
        trg = 4 * cid + tr
        row0 = pl.multiple_of(8 * trg, 8)

        def zblk(j, carry):
            p, q0 = pq(j)
            for r in range(8):
                acc[r, p, pl.ds(q0, _L)] = zero16
            return carry

        lax.fori_loop(0, 128, zblk, 0)

        def field_body(f, carry):
            delta = 32 * (f % 4)
            coff = f * _V - delta
            choff = pl.multiple_of(coff + k * _CHK, 128)
            pltpu.sync_copy(
                x3_hbm.at[f, pl.ds(pl.multiple_of(bh * 16, 8), 16)], xfv)
            pltpu.sync_copy(wt_hbm.at[pl.ds(row0, 8), pl.ds(choff, _CHK)], buf)
            shift = delta - k * _CHK

            def blk(j, c2):
                return gather_block(j, shift, 0, _CHK, buf, c2)

            lax.fori_loop(0, 128, blk, 0)
            return carry

        lax.fori_loop(0, _F - 1, field_body, 0)

        f25 = _F - 1
        delta25 = f25 * _V - 2499968
        choff25 = pl.multiple_of(
            jnp.where(k == 7, _LAST7, 2499968 + k * _CHK), 128)
        lo25 = jnp.where(k == 7, (2499968 + 7 * _CHK) - _LAST7, 0)
        pltpu.sync_copy(
            x3_hbm.at[f25, pl.ds(pl.multiple_of(bh * 16, 8), 16)], xfv)
        pltpu.sync_copy(wt_hbm.at[pl.ds(row0, 8), pl.ds(choff25, _CHK)], buf)
        sh25 = delta25 - (choff25 - 2499968)

        def blk25(j, c2):
            return gather_block(j, sh25, lo25, _CHK, buf, c2)

        lax.fori_loop(0, 128, blk25, 0)

        @pl.when(k == 0)
        def _():
            pltpu.sync_copy(tail4_hbm.at[trg], tailv)

            def blkt(j, c2):
                p, q0 = pq(j)
                xa = xfv[p, pl.ds(q0, _L)]
                idxl = xa - (_TAIL0 - f25 * _V)
                valid = idxl >= 0
                idc = jnp.clip(idxl, 0, _N - _TAIL0 - 1)
                for r in range(8):
                    rvec = jnp.full((_L,), r, dtype=jnp.int32)
                    v = plsc.load_gather(tailv, [rvec, idc])
                    vm = jnp.where(valid, v, 0.0)
                    plsc.addupdate(acc.at[r, p, pl.ds(q0, _L)], vm)
                return c2

            lax.fori_loop(0, 128, blkt, 0)

        for rg in range(4):
            for rr in range(2):
                pltpu.sync_copy(acc.at[2 * rg + rr], mbuf.at[k, bh, rr])
            plsc.subcore_barrier()

            @pl.when(k < 2)
            def _():
                c = 8 * trg + 2 * rg + k
                cvec = jnp.full((_L,), c, dtype=jnp.int32)
                bval = plsc.load_gather(biasv, [cvec])

                def initm(j, carry):
                    p, q0 = pq(j)
                    macc[p, pl.ds(q0, _L)] = bval
                    return carry

                lax.fori_loop(0, 128, initm, 0)

                for kk in range(8):
                    pltpu.sync_copy(mbuf.at[kk, bh, k], tmp)

                    def madd(j, carry):
                        p, q0 = pq(j)
                        sl = pl.ds(q0, _L)
                        macc[p, sl] = macc[p, sl] + tmp[p, sl]
                        return carry

                    lax.fori_loop(0, 128, madd, 0)

                pltpu.sync_copy(macc, out5_hbm.at[trg, 2 * rg + k, bh])

            plsc.subcore_barrier()


@jax.jit
def _fieldlinear_sc(x3, wt, bias, tail4):
    mesh = plsc.VectorSubcoreMesh(core_axis_name="c", subcore_axis_name="s")
    kern = functools.partial(
        pl.kernel,
        out_type=jax.ShapeDtypeStruct((8, 8, 2, 16, 128), jnp.float32),
        mesh=mesh,
        compiler_params=pltpu.CompilerParams(needs_layout_passes=False),
        scratch_types=[
            pltpu.VMEM((16, 128), jnp.int32),
            pltpu.VMEM((8, _CHK), jnp.float32),
            pltpu.VMEM((8, 16, 128), jnp.float32),
            pltpu.VMEM((16, 128), jnp.float32),
            pltpu.VMEM((16, 128), jnp.float32),
            pltpu.VMEM((8, 64), jnp.float32),
            pltpu.VMEM((_D,), jnp.float32),
            pltpu.VMEM_SHARED((8, 2, 2, 16, 128), jnp.float32),
        ],
    )(_body)
    return kern(x3, wt, bias, tail4)


def kernel(x, weight, bias):
    x3 = x.T.reshape(_F, 32, 128)
    tail4 = weight[_TAIL0:].T.reshape(8, 8, _N - _TAIL0)
    o5 = _fieldlinear_sc(x3, weight.T, bias, tail4)
    return o5.reshape(_D, _B).T


# speedup vs baseline: 1.4114x; 1.2044x over previous
import functools

import jax
import jax.numpy as jnp
from jax import lax
from jax.experimental import pallas as pl
from jax.experimental.pallas import tpu as pltpu
from jax.experimental.pallas import tpu_sc as plsc

_B = 4096
_F = 26
_D = 64
_V = 100000
_N = _F * _V
_L = 16
_CHK = 12544
_HALF = _CHK // 2
_TAIL0 = 2599936
_LAST7 = _TAIL0 - _CHK
_LO7 = (2499968 + 7 * _CHK) - _LAST7


def _body(x3_hbm, wt_hbm, bias_hbm, tail4_hbm, out5_hbm, part_hbm,
          xv0, xv1, bufa, bufb, acc, macc, tailv, biasv,
          sema, semb, semx, semp):
    cid = lax.axis_index("c")
    sid = lax.axis_index("s")
    k = sid % 8
    bh = sid // 8

    pltpu.sync_copy(bias_hbm, biasv)
    xrow = pl.multiple_of(bh * 16, 8)

    zero16 = jnp.zeros((_L,), jnp.float32)

    def choff_of(f):
        delta = 32 * (f % 4)
        base = f * _V - delta + k * _CHK
        return pl.multiple_of(
            jnp.where((f == _F - 1) & (k == 7), _LAST7, base), 128)

    def tr_body(tr, tr_carry):
        trg = 4 * cid + tr
        row0 = pl.multiple_of(8 * trg, 8)

        def zblk(p, carry):
            for qi in range(8):
                q0 = qi * _L
                for r in range(8):
                    acc[r, p, pl.ds(q0, _L)] = zero16
            return carry

        lax.fori_loop(0, 16, zblk, 0)

        def issue(f, h, buf, sem):
            choff = pl.multiple_of(choff_of(f) + h * _HALF, 128)
            pltpu.async_copy(
                wt_hbm.at[pl.ds(row0, 8), pl.ds(choff, _HALF)], buf, sem)

        def issue_x(f, xv, sem):
            pltpu.async_copy(
                x3_hbm.at[f, pl.ds(xrow, 16)], xv, sem)

        def wait(buf, sem):
            pltpu.make_async_copy(
                wt_hbm.at[pl.ds(0, 8), pl.ds(0, _HALF)], buf, sem).wait()

        def wait_x(xv, sem):
            pltpu.make_async_copy(
                x3_hbm.at[0, pl.ds(0, 16)], xv, sem).wait()

        def scan(f, h, buf, xv, lo):
            shift = f * _V - (choff_of(f) + h * _HALF)

            def prow(p, carry):
                for qi in range(8):
                    q0 = qi * _L
                    xa = xv[p, pl.ds(q0, _L)]
                    idxl = xa + shift
                    valid = (idxl >= lo) & (idxl < _HALF)
                    idc = jnp.clip(idxl, 0, _HALF - 1)
                    for r in range(8):
                        rvec = jnp.full((_L,), r, dtype=jnp.int32)
                        v = plsc.load_gather(buf, [rvec, idc])
                        vm = jnp.where(valid, v, 0.0)
                        plsc.addupdate(acc.at[r, p, pl.ds(q0, _L)], vm)
                return carry

            lax.fori_loop(0, 16, prow, 0)

        issue_x(0, xv0, semx)
        issue(0, 0, bufa, sema)
        issue(0, 1, bufb, semb)
        wait_x(xv0, semx)

        def f2_body(f2, carry):
            for ff in range(2):
                f = 2 * f2 + ff
                xv = xv0 if ff == 0 else xv1
                xvn = xv1 if ff == 0 else xv0
                lo0 = jnp.where((f == _F - 1) & (k == 7), _LO7, 0)

                @pl.when(f < _F - 1)
                def _():
                    issue_x(f + 1, xvn, semx)

                wait(bufa, sema)
                scan(f, 0, bufa, xv, lo0)

                @pl.when(f < _F - 1)
                def _():
                    issue(f + 1, 0, bufa, sema)

                wait(bufb, semb)
                scan(f, 1, bufb, xv, 0)

                @pl.when(f < _F - 1)
                def _():
                    issue(f + 1, 1, bufb, semb)
                    wait_x(xvn, semx)

            return carry

        lax.fori_loop(0, _F // 2, f2_body, 0)

        @pl.when(k == 0)
        def _():
            pltpu.sync_copy(tail4_hbm.at[trg], tailv)

            def blkt(p, c2):
                for qi in range(8):
                    q0 = qi * _L
                    xa = xv1[p, pl.ds(q0, _L)]
                    idxl = xa - (_TAIL0 - (_F - 1) * _V)
                    valid = idxl >= 0
                    idc = jnp.clip(idxl, 0, _N - _TAIL0 - 1)
                    for r in range(8):
                        rvec = jnp.full((_L,), r, dtype=jnp.int32)
                        v = plsc.load_gather(tailv, [rvec, idc])
                        vm = jnp.where(valid, v, 0.0)
                        plsc.addupdate(acc.at[r, p, pl.ds(q0, _L)], vm)
                return c2

            lax.fori_loop(0, 16, blkt, 0)

        for r in range(8):
            pltpu.async_copy(acc.at[r], part_hbm.at[trg, bh, r, k], semp)
        pltpu.make_async_copy(part_hbm.at[trg, bh, 0], acc, semp).wait()
        plsc.subcore_barrier()

        pltpu.sync_copy(part_hbm.at[trg, bh, k], acc)
        c = 8 * trg + k
        cvec = jnp.full((_L,), c, dtype=jnp.int32)
        bval = plsc.load_gather(biasv, [cvec])

        def merge(p, carry):
            for qi in range(8):
                q0 = qi * _L
                sl = pl.ds(q0, _L)
                s = acc[0, p, sl] + bval
                for kk in range(1, 8):
                    s = s + acc[kk, p, sl]
                macc[p, sl] = s
            return carry

        lax.fori_loop(0, 16, merge, 0)
        pltpu.sync_copy(macc, out5_hbm.at[trg, k, bh])
        plsc.subcore_barrier()
        return tr_carry

    lax.fori_loop(0, 4, tr_body, 0)


@jax.jit
def _fieldlinear_sc(x3, wt, bias, tail4):
    mesh = plsc.VectorSubcoreMesh(core_axis_name="c", subcore_axis_name="s")
    kern = functools.partial(
        pl.kernel,
        out_type=(
            jax.ShapeDtypeStruct((8, 8, 2, 16, 128), jnp.float32),
            jax.ShapeDtypeStruct((8, 2, 8, 8, 16, 128), jnp.float32),
        ),
        mesh=mesh,
        compiler_params=pltpu.CompilerParams(needs_layout_passes=False),
        scratch_types=[
            pltpu.VMEM((16, 128), jnp.int32),
            pltpu.VMEM((16, 128), jnp.int32),
            pltpu.VMEM((8, _HALF), jnp.float32),
            pltpu.VMEM((8, _HALF), jnp.float32),
            pltpu.VMEM((8, 16, 128), jnp.float32),
            pltpu.VMEM((16, 128), jnp.float32),
            pltpu.VMEM((8, 64), jnp.float32),
            pltpu.VMEM((_D,), jnp.float32),
            pltpu.SemaphoreType.DMA,
            pltpu.SemaphoreType.DMA,
            pltpu.SemaphoreType.DMA,
            pltpu.SemaphoreType.DMA,
        ],
    )(_body)
    return kern(x3, wt, bias, tail4)


def kernel(x, weight, bias):
    x3 = x.T.reshape(_F, 32, 128)
    tail4 = weight[_TAIL0:].T.reshape(8, 8, _N - _TAIL0)
    o5, _part = _fieldlinear_sc(x3, weight.T, bias, tail4)
    return o5.reshape(_D, _B).T


# speedup vs baseline: 2.6509x; 1.8782x over previous
import functools

import jax
import jax.numpy as jnp
from jax import lax
from jax.experimental import pallas as pl
from jax.experimental.pallas import tpu as pltpu
from jax.experimental.pallas import tpu_sc as plsc

_B = 4096
_F = 26
_D = 64
_V = 100000
_N = _F * _V
_L = 16
_CHK = 12544
_HALF = _CHK // 2
_TAIL0 = 2599936
_LAST7 = _TAIL0 - _CHK
_LO7 = (2499968 + 7 * _CHK) - _LAST7


def _body(x3_hbm, wt_hbm, bias_hbm, tail4_hbm, out5_hbm, part_hbm,
          xv0, xv1, bufa, bufb, acc, macc, tailv, biasv,
          sema, semb, semx, semp):
    cid = lax.axis_index("c")
    sid = lax.axis_index("s")
    k = sid % 8
    bh = sid // 8

    pltpu.sync_copy(bias_hbm, biasv)
    xrow = pl.multiple_of(bh * 16, 8)

    zero16 = jnp.zeros((_L,), jnp.float32)

    def choff_of(f):
        delta = 32 * (f % 4)
        base = f * _V - delta + k * _CHK
        return pl.multiple_of(
            jnp.where((f == _F - 1) & (k == 7), _LAST7, base), 128)

    def tr_body(tr, tr_carry):
        trg = 4 * cid + tr
        row0 = pl.multiple_of(8 * trg, 8)

        @plsc.parallel_loop(0, 16)
        def zblk(p):
            for qi in range(8):
                q0 = qi * _L
                for r in range(8):
                    acc[r, p, pl.ds(q0, _L)] = zero16

        def issue(f, h, buf, sem):
            choff = pl.multiple_of(choff_of(f) + h * _HALF, 128)
            pltpu.async_copy(
                wt_hbm.at[pl.ds(row0, 8), pl.ds(choff, _HALF)], buf, sem)

        def issue_x(f, xv, sem):
            pltpu.async_copy(
                x3_hbm.at[f, pl.ds(xrow, 16)], xv, sem)

        def wait(buf, sem):
            pltpu.make_async_copy(
                wt_hbm.at[pl.ds(0, 8), pl.ds(0, _HALF)], buf, sem).wait()

        def wait_x(xv, sem):
            pltpu.make_async_copy(
                x3_hbm.at[0, pl.ds(0, 16)], xv, sem).wait()

        def scan(f, h, buf, xv, lo):
            shift = f * _V - (choff_of(f) + h * _HALF)

            @plsc.parallel_loop(0, 16)
            def prow(p):
                for qi in range(8):
                    q0 = qi * _L
                    xa = xv[p, pl.ds(q0, _L)]
                    idxl = xa + shift
                    valid = (idxl >= lo) & (idxl < _HALF)
                    idc = jnp.clip(idxl, 0, _HALF - 1)
                    for r in range(8):
                        rvec = jnp.full((_L,), r, dtype=jnp.int32)
                        v = plsc.load_gather(buf, [rvec, idc])
                        vm = jnp.where(valid, v, 0.0)
                        plsc.addupdate(acc.at[r, p, pl.ds(q0, _L)], vm)

        issue_x(0, xv0, semx)
        issue(0, 0, bufa, sema)
        issue(0, 1, bufb, semb)
        wait_x(xv0, semx)

        def f2_body(f2, carry):
            for ff in range(2):
                f = 2 * f2 + ff
                xv = xv0 if ff == 0 else xv1
                xvn = xv1 if ff == 0 else xv0
                lo0 = jnp.where((f == _F - 1) & (k == 7), _LO7, 0)

                @pl.when(f < _F - 1)
                def _():
                    issue_x(f + 1, xvn, semx)

                wait(bufa, sema)
                scan(f, 0, bufa, xv, lo0)

                @pl.when(f < _F - 1)
                def _():
                    issue(f + 1, 0, bufa, sema)

                wait(bufb, semb)
                scan(f, 1, bufb, xv, 0)

                @pl.when(f < _F - 1)
                def _():
                    issue(f + 1, 1, bufb, semb)
                    wait_x(xvn, semx)

            return carry

        lax.fori_loop(0, _F // 2, f2_body, 0)

        @pl.when(k == 0)
        def _():
            pltpu.sync_copy(tail4_hbm.at[trg], tailv)

            @plsc.parallel_loop(0, 16)
            def blkt(p):
                for qi in range(8):
                    q0 = qi * _L
                    xa = xv1[p, pl.ds(q0, _L)]
                    idxl = xa - (_TAIL0 - (_F - 1) * _V)
                    valid = idxl >= 0
                    idc = jnp.clip(idxl, 0, _N - _TAIL0 - 1)
                    for r in range(8):
                        rvec = jnp.full((_L,), r, dtype=jnp.int32)
                        v = plsc.load_gather(tailv, [rvec, idc])
                        vm = jnp.where(valid, v, 0.0)
                        plsc.addupdate(acc.at[r, p, pl.ds(q0, _L)], vm)

        for r in range(8):
            pltpu.async_copy(acc.at[r], part_hbm.at[trg, bh, r, k], semp)
        pltpu.make_async_copy(part_hbm.at[trg, bh, 0], acc, semp).wait()
        plsc.subcore_barrier()

        pltpu.sync_copy(part_hbm.at[trg, bh, k], acc)
        c = 8 * trg + k
        cvec = jnp.full((_L,), c, dtype=jnp.int32)
        bval = plsc.load_gather(biasv, [cvec])

        @plsc.parallel_loop(0, 16)
        def merge(p):
            for qi in range(8):
                q0 = qi * _L
                sl = pl.ds(q0, _L)
                s = acc[0, p, sl] + bval
                for kk in range(1, 8):
                    s = s + acc[kk, p, sl]
                macc[p, sl] = s
        pltpu.sync_copy(macc, out5_hbm.at[trg, k, bh])
        plsc.subcore_barrier()
        return tr_carry

    lax.fori_loop(0, 4, tr_body, 0)


@jax.jit
def _fieldlinear_sc(x3, wt, bias, tail4):
    mesh = plsc.VectorSubcoreMesh(core_axis_name="c", subcore_axis_name="s")
    kern = functools.partial(
        pl.kernel,
        out_type=(
            jax.ShapeDtypeStruct((8, 8, 2, 16, 128), jnp.float32),
            jax.ShapeDtypeStruct((8, 2, 8, 8, 16, 128), jnp.float32),
        ),
        mesh=mesh,
        compiler_params=pltpu.CompilerParams(needs_layout_passes=False),
        scratch_types=[
            pltpu.VMEM((16, 128), jnp.int32),
            pltpu.VMEM((16, 128), jnp.int32),
            pltpu.VMEM((8, _HALF), jnp.float32),
            pltpu.VMEM((8, _HALF), jnp.float32),
            pltpu.VMEM((8, 16, 128), jnp.float32),
            pltpu.VMEM((16, 128), jnp.float32),
            pltpu.VMEM((8, 64), jnp.float32),
            pltpu.VMEM((_D,), jnp.float32),
            pltpu.SemaphoreType.DMA,
            pltpu.SemaphoreType.DMA,
            pltpu.SemaphoreType.DMA,
            pltpu.SemaphoreType.DMA,
        ],
    )(_body)
    return kern(x3, wt, bias, tail4)


def kernel(x, weight, bias):
    x3 = x.T.reshape(_F, 32, 128)
    tail4 = weight[_TAIL0:].T.reshape(8, 8, _N - _TAIL0)
    o5, _part = _fieldlinear_sc(x3, weight.T, bias, tail4)
    return o5.reshape(_D, _B).T


# speedup vs baseline: 2.7358x; 1.0320x over previous
import functools

import jax
import jax.numpy as jnp
from jax import lax
from jax.experimental import pallas as pl
from jax.experimental.pallas import tpu as pltpu
from jax.experimental.pallas import tpu_sc as plsc

_B = 4096
_F = 26
_D = 64
_V = 100000
_N = _F * _V
_L = 16
_CHK = 12544
_HALF = _CHK // 2
_TAIL0 = 2599936
_LAST7 = _TAIL0 - _CHK
_LO7 = (2499968 + 7 * _CHK) - _LAST7


def _body(x3_hbm, wt_hbm, bias_hbm, tail4_hbm, out5_hbm, part_hbm,
          xv0, xv1, bufa, bufb, acc, macc, tailv, biasv,
          sema, semb, semx, semp):
    cid = lax.axis_index("c")
    sid = lax.axis_index("s")
    k = sid % 8
    bh = sid // 8

    pltpu.sync_copy(bias_hbm, biasv)
    xrow = pl.multiple_of(bh * 16, 8)

    zero16 = jnp.zeros((_L,), jnp.float32)

    def choff_of(f):
        delta = 32 * (f % 4)
        base = f * _V - delta + k * _CHK
        return pl.multiple_of(
            jnp.where((f == _F - 1) & (k == 7), _LAST7, base), 128)

    for _pb in (bufa, bufb):
        for _r in range(8):
            _pb[_r, pl.ds(_HALF, _L)] = zero16

    def tr_body(tr, tr_carry):
        trg = 4 * cid + tr
        row0 = pl.multiple_of(8 * trg, 8)

        @plsc.parallel_loop(0, 16)
        def zblk(p):
            for qi in range(8):
                q0 = qi * _L
                for r in range(8):
                    acc[r, p, pl.ds(q0, _L)] = zero16

        def issue(f, h, buf, sem):
            choff = pl.multiple_of(choff_of(f) + h * _HALF, 128)
            pltpu.async_copy(
                wt_hbm.at[pl.ds(row0, 8), pl.ds(choff, _HALF)],
                buf.at[:, pl.ds(0, _HALF)], sem)

        def issue_x(f, xv, sem):
            pltpu.async_copy(
                x3_hbm.at[f, pl.ds(xrow, 16)], xv, sem)

        def wait(buf, sem):
            pltpu.make_async_copy(
                wt_hbm.at[pl.ds(0, 8), pl.ds(0, _HALF)],
                buf.at[:, pl.ds(0, _HALF)], sem).wait()

        def wait_x(xv, sem):
            pltpu.make_async_copy(
                x3_hbm.at[0, pl.ds(0, 16)], xv, sem).wait()

        def scan(f, h, buf, xv, lo):
            shift = f * _V - (choff_of(f) + h * _HALF)

            @plsc.parallel_loop(0, 16)
            def prow(p):
                for qi in range(8):
                    q0 = qi * _L
                    xa = xv[p, pl.ds(q0, _L)]
                    idm = jnp.minimum(
                        (xa + (shift - lo)).astype(jnp.uint32),
                        jnp.uint32(_HALF) - lo.astype(jnp.uint32))
                    idc = idm.astype(jnp.int32) + lo
                    for r in range(8):
                        rvec = jnp.full((_L,), r, dtype=jnp.int32)
                        v = plsc.load_gather(buf, [rvec, idc])
                        plsc.addupdate(acc.at[r, p, pl.ds(q0, _L)], v)

        issue_x(0, xv0, semx)
        issue(0, 0, bufa, sema)
        issue(0, 1, bufb, semb)
        wait_x(xv0, semx)

        def f2_body(f2, carry):
            for ff in range(2):
                f = 2 * f2 + ff
                xv = xv0 if ff == 0 else xv1
                xvn = xv1 if ff == 0 else xv0
                lo0 = jnp.where((f == _F - 1) & (k == 7), _LO7, 0)

                @pl.when(f < _F - 1)
                def _():
                    issue_x(f + 1, xvn, semx)

                wait(bufa, sema)
                scan(f, 0, bufa, xv, lo0)

                @pl.when(f < _F - 1)
                def _():
                    issue(f + 1, 0, bufa, sema)

                wait(bufb, semb)
                scan(f, 1, bufb, xv, jnp.int32(0))

                @pl.when(f < _F - 1)
                def _():
                    issue(f + 1, 1, bufb, semb)
                    wait_x(xvn, semx)

            return carry

        lax.fori_loop(0, _F // 2, f2_body, 0)

        @pl.when(k == 0)
        def _():
            pltpu.sync_copy(tail4_hbm.at[trg], tailv)

            @plsc.parallel_loop(0, 16)
            def blkt(p):
                for qi in range(8):
                    q0 = qi * _L
                    xa = xv1[p, pl.ds(q0, _L)]
                    idxl = xa - (_TAIL0 - (_F - 1) * _V)
                    valid = idxl >= 0
                    idc = jnp.clip(idxl, 0, _N - _TAIL0 - 1)
                    for r in range(8):
                        rvec = jnp.full((_L,), r, dtype=jnp.int32)
                        v = plsc.load_gather(tailv, [rvec, idc])
                        vm = jnp.where(valid, v, 0.0)
                        plsc.addupdate(acc.at[r, p, pl.ds(q0, _L)], vm)

        for r in range(8):
            pltpu.async_copy(acc.at[r], part_hbm.at[trg, bh, r, k], semp)
        pltpu.make_async_copy(part_hbm.at[trg, bh, 0], acc, semp).wait()
        plsc.subcore_barrier()

        pltpu.sync_copy(part_hbm.at[trg, bh, k], acc)
        c = 8 * trg + k
        cvec = jnp.full((_L,), c, dtype=jnp.int32)
        bval = plsc.load_gather(biasv, [cvec])

        @plsc.parallel_loop(0, 16)
        def merge(p):
            for qi in range(8):
                q0 = qi * _L
                sl = pl.ds(q0, _L)
                s = acc[0, p, sl] + bval
                for kk in range(1, 8):
                    s = s + acc[kk, p, sl]
                macc[p, sl] = s
        pltpu.sync_copy(macc, out5_hbm.at[trg, k, bh])
        plsc.subcore_barrier()
        return tr_carry

    lax.fori_loop(0, 4, tr_body, 0)


@jax.jit
def _fieldlinear_sc(x3, wt, bias, tail4):
    mesh = plsc.VectorSubcoreMesh(core_axis_name="c", subcore_axis_name="s")
    kern = functools.partial(
        pl.kernel,
        out_type=(
            jax.ShapeDtypeStruct((8, 8, 2, 16, 128), jnp.float32),
            jax.ShapeDtypeStruct((8, 2, 8, 8, 16, 128), jnp.float32),
        ),
        mesh=mesh,
        compiler_params=pltpu.CompilerParams(needs_layout_passes=False),
        scratch_types=[
            pltpu.VMEM((16, 128), jnp.int32),
            pltpu.VMEM((16, 128), jnp.int32),
            pltpu.VMEM((8, _HALF + 128), jnp.float32),
            pltpu.VMEM((8, _HALF + 128), jnp.float32),
            pltpu.VMEM((8, 16, 128), jnp.float32),
            pltpu.VMEM((16, 128), jnp.float32),
            pltpu.VMEM((8, 64), jnp.float32),
            pltpu.VMEM((_D,), jnp.float32),
            pltpu.SemaphoreType.DMA,
            pltpu.SemaphoreType.DMA,
            pltpu.SemaphoreType.DMA,
            pltpu.SemaphoreType.DMA,
        ],
    )(_body)
    return kern(x3, wt, bias, tail4)


def kernel(x, weight, bias):
    x3 = x.T.reshape(_F, 32, 128)
    tail4 = weight[_TAIL0:].T.reshape(8, 8, _N - _TAIL0)
    o5, _part = _fieldlinear_sc(x3, weight.T, bias, tail4)
    return o5.reshape(_D, _B).T
